# Initial kernel scaffold; baseline (speedup 1.0000x reference)
import jax
import jax.numpy as jnp
from jax.experimental import pallas as pl


def kernel(xA, edge_indexA, edge_attrA, xB, edge_indexB, edge_attrB, W_type, W1, b1, W2, b2, gamma, beta):
    raise NotImplementedError("write your pallas kernel here")


# speedup vs baseline: 2.9281x; 2.9281x over previous
import functools

import jax
import jax.numpy as jnp
from jax import lax
from jax.experimental import pallas as pl
from jax.experimental.pallas import tpu as pltpu
from jax.experimental.pallas import tpu_sc as plsc

N = 10000
D = 256
E = 160000
NT = 5
K = 64
EPAD = 171008
EROWS = EPAD // K
BPT = EPAD // (16 * K)
HALF = N // 2
ACC_ROWS = 5024
ZCHUNK = ACC_ROWS // 16
TRASH = 5008
WCHUNK = 313
PADDST = 3 * N
EPS = 1e-5
MB = 1000


def _idx_body(src_ref, typ_ref, g_ref):
    g_ref[...] = typ_ref[...] * N + src_ref[...]


def _idx_prep(srcp, typp):
    grid = EROWS // 8
    return pl.pallas_call(
        _idx_body,
        grid=(grid,),
        in_specs=[pl.BlockSpec((8, K), lambda i: (i, 0))] * 2,
        out_specs=pl.BlockSpec((8, K), lambda i: (i, 0)),
        out_shape=jax.ShapeDtypeStruct((EROWS, K), jnp.int32),
    )(srcp, typp)


def _ybuild_body(x_ref, w_ref, y_ref):
    t = pl.program_id(0)
    w = w_ref[pl.ds(t, 1), :]
    y_ref[0] = jnp.maximum(x_ref[...] + w, 0.0)


def _ybuild(x, w_type):
    yb = 1000
    return pl.pallas_call(
        _ybuild_body,
        grid=(NT, N // yb),
        in_specs=[
            pl.BlockSpec((yb, D), lambda t, i: (i, 0)),
            pl.BlockSpec((8, D), lambda t, i: (0, 0)),
        ],
        out_specs=pl.BlockSpec((1, yb, D), lambda t, i: (t, i, 0)),
        out_shape=jax.ShapeDtypeStruct((NT, N, D), jnp.float32),
    )(x, w_type).reshape(NT * N, D)


_SC_MESH = plsc.VectorSubcoreMesh(core_axis_name="c", subcore_axis_name="s")


@functools.partial(
    pl.kernel,
    mesh=_SC_MESH,
    out_type=jax.ShapeDtypeStruct((N, D), jnp.float32),
    scratch_types=[
        pltpu.VMEM((BPT, K), jnp.int32),
        pltpu.VMEM((BPT, K), jnp.int32),
        pltpu.VMEM((K, D), jnp.float32),
        pltpu.VMEM_SHARED((ACC_ROWS, D), jnp.float32),
        pltpu.SemaphoreType.DMA,
    ],
    compiler_params=pltpu.CompilerParams(use_tc_tiling_on_sc=False),
)
def _sc_agg(y_hbm, gidx_hbm, dst_hbm, out_hbm,
            gix_v, dix_v, rows_v, acc_sh, sem):
    cid = lax.axis_index("c")
    sid = lax.axis_index("s")

    def zbody(j, c):
        rows_v[j // 16, pl.ds((j % 16) * 16, 16)] = jnp.zeros((16,), jnp.float32)
        return c

    lax.fori_loop(0, K * 16, zbody, 0)
    zb = sid * ZCHUNK
    for o in range(0, ZCHUNK - K + 1, K):
        pltpu.sync_copy(rows_v, acc_sh.at[pl.ds(zb + o, K)])
    rem = ZCHUNK % K
    if rem:
        pltpu.sync_copy(rows_v.at[pl.ds(0, rem)],
                        acc_sh.at[pl.ds(zb + ZCHUNK - rem, rem)])
    pltpu.sync_copy(gidx_hbm.at[pl.ds(sid * BPT, BPT)], gix_v)
    pltpu.sync_copy(dst_hbm.at[pl.ds(sid * BPT, BPT)], dix_v)

    base = cid * HALF

    nv = K // 16

    def lbody(j, c):
        sl = (j // nv, pl.ds((j % nv) * 16, 16))
        v = dix_v[sl]
        inr = (v >= base) & (v < base + HALF)
        dix_v[sl] = jnp.where(inr, v - base, TRASH)
        return c

    lax.fori_loop(0, BPT * nv, lbody, 0)
    plsc.subcore_barrier()

    def body(b, c):
        pltpu.async_copy(y_hbm.at[gix_v.at[b]], rows_v, sem).wait()
        pltpu.sync_copy(rows_v, acc_sh.at[dix_v.at[b]], add=True)
        return c

    lax.fori_loop(0, BPT, body, 0)
    plsc.subcore_barrier()
    start = jnp.minimum(sid * WCHUNK, HALF - WCHUNK)
    pltpu.sync_copy(acc_sh.at[pl.ds(start, WCHUNK)],
                    out_hbm.at[pl.ds(cid * HALF + start, WCHUNK)])


def _mlp_body(a_ref, w1_ref, b1_ref, w2_ref, b2_ref, h_ref, st_ref):
    i = pl.program_id(0)
    a = a_ref[...]
    h1 = lax.dot_general(a, w1_ref[...], (((1,), (1,)), ((), ())),
                         precision=lax.Precision.HIGHEST,
                         preferred_element_type=jnp.float32)
    h1 = jnp.maximum(h1 + b1_ref[...], 0.0)
    h = lax.dot_general(h1, w2_ref[...], (((1,), (1,)), ((), ())),
                        precision=lax.Precision.HIGHEST,
                        preferred_element_type=jnp.float32)
    h = h + b2_ref[...]
    h_ref[...] = h

    @pl.when(i == 0)
    def _():
        st_ref[...] = jnp.zeros_like(st_ref)

    st_ref[0:1, :] += jnp.sum(h, axis=0, keepdims=True)
    st_ref[1:2, :] += jnp.sum(h * h, axis=0, keepdims=True)


def _mlp(aggr, w1, b1, w2, b2):
    return pl.pallas_call(
        _mlp_body,
        grid=(N // MB,),
        in_specs=[
            pl.BlockSpec((MB, D), lambda i: (i, 0)),
            pl.BlockSpec((2 * D, D), lambda i: (0, 0)),
            pl.BlockSpec((1, 2 * D), lambda i: (0, 0)),
            pl.BlockSpec((D, 2 * D), lambda i: (0, 0)),
            pl.BlockSpec((1, D), lambda i: (0, 0)),
        ],
        out_specs=[
            pl.BlockSpec((MB, D), lambda i: (i, 0)),
            pl.BlockSpec((8, D), lambda i: (0, 0)),
        ],
        out_shape=[
            jax.ShapeDtypeStruct((N, D), jnp.float32),
            jax.ShapeDtypeStruct((8, D), jnp.float32),
        ],
    )(aggr, w1, b1, w2, b2)


def _norm_body(h_ref, st_ref, gam_ref, bet_ref, o_ref):
    mean = st_ref[0:1, :] * (1.0 / N)
    var = st_ref[1:2, :] * (1.0 / N) - mean * mean
    inv = lax.rsqrt(var + EPS)
    o_ref[...] = jnp.maximum(
        (h_ref[...] - mean) * inv * gam_ref[...] + bet_ref[...], 0.0)


def _norm(h, st, gamma, beta):
    return pl.pallas_call(
        _norm_body,
        grid=(N // MB,),
        in_specs=[
            pl.BlockSpec((MB, D), lambda i: (i, 0)),
            pl.BlockSpec((8, D), lambda i: (0, 0)),
            pl.BlockSpec((1, D), lambda i: (0, 0)),
            pl.BlockSpec((1, D), lambda i: (0, 0)),
        ],
        out_specs=pl.BlockSpec((MB, D), lambda i: (i, 0)),
        out_shape=jax.ShapeDtypeStruct((N, D), jnp.float32),
    )(h, st, gamma, beta)


def _process(x, edge_index, edge_attr, w_type, w1, b1, w2, b2, gamma, beta):
    ei = edge_index.astype(jnp.int32)
    t = edge_attr[:, 0].astype(jnp.int32)
    loops = jnp.arange(N, dtype=jnp.int32)
    npad = EPAD - E - N
    padz = jnp.zeros((npad,), jnp.int32)
    srcp = jnp.concatenate([ei[0], loops, padz]).reshape(EROWS, K)
    dstp = jnp.concatenate(
        [ei[1], loops, jnp.full((npad,), PADDST, jnp.int32)]).reshape(EROWS, K)
    typp = jnp.concatenate(
        [t, jnp.full((N,), 4, jnp.int32), padz]).reshape(EROWS, K)
    gidx = _idx_prep(srcp, typp)
    y = _ybuild(x, w_type)
    aggr = _sc_agg(y, gidx, dstp)
    h, st = _mlp(aggr, w1, b1.reshape(1, -1), w2, b2.reshape(1, -1))
    return _norm(h, st, gamma.reshape(1, -1), beta.reshape(1, -1))


def kernel(xA, edge_indexA, edge_attrA, xB, edge_indexB, edge_attrB,
           W_type, W1, b1, W2, b2, gamma, beta):
    outA = _process(xA, edge_indexA, edge_attrA, W_type, W1, b1, W2, b2,
                    gamma, beta)
    outB = _process(xB, edge_indexB, edge_attrB, W_type, W1, b1, W2, b2,
                    gamma, beta)
    return (outA, outB)


# speedup vs baseline: 3.1825x; 1.0869x over previous
import functools

import jax
import jax.numpy as jnp
from jax import lax
from jax.experimental import pallas as pl
from jax.experimental.pallas import tpu as pltpu
from jax.experimental.pallas import tpu_sc as plsc

N = 10000
D = 256
E = 160000
NT = 5
K = 64
EPAD = 172032
EROWS = EPAD // K
BPT = EPAD // (16 * K)
HALF = N // 2
ACC_ROWS = 5024
ZCHUNK = ACC_ROWS // 16
TRASH = 5008
WCHUNK = 313
PADDST = 2 * N
EPS = 1e-5
MB = 1000


def _idx_body(src_ref, dst_ref, typ_ref, p_ref):
    p_ref[...] = dst_ref[...] * 65536 + (typ_ref[...] * N + src_ref[...])


def _idx_prep(srcp, dstp, typp):
    grid = EROWS // 8
    return pl.pallas_call(
        _idx_body,
        grid=(grid,),
        in_specs=[pl.BlockSpec((8, K), lambda i: (i, 0))] * 3,
        out_specs=pl.BlockSpec((8, K), lambda i: (i, 0)),
        out_shape=jax.ShapeDtypeStruct((EROWS, K), jnp.int32),
    )(srcp, dstp, typp)


def _ybuild_body(x_ref, w_ref, y_ref):
    t = pl.program_id(0)
    w = w_ref[pl.ds(t, 1), :]
    y_ref[...] = jnp.maximum(x_ref[...] + w, 0.0)


def _ybuild(x, w_type):
    yb = 1000
    nb = N // yb
    return pl.pallas_call(
        _ybuild_body,
        grid=(NT, nb),
        in_specs=[
            pl.BlockSpec((yb, D), lambda t, i: (i, 0)),
            pl.BlockSpec((8, D), lambda t, i: (0, 0)),
        ],
        out_specs=pl.BlockSpec((yb, D), lambda t, i: (t * nb + i, 0)),
        out_shape=jax.ShapeDtypeStruct((NT * N, D), jnp.float32),
    )(x, w_type)


_SC_MESH = plsc.VectorSubcoreMesh(core_axis_name="c", subcore_axis_name="s")


@functools.partial(
    pl.kernel,
    mesh=_SC_MESH,
    out_type=jax.ShapeDtypeStruct((N, D), jnp.float32),
    scratch_types=[
        pltpu.VMEM((BPT, K), jnp.int32),
        pltpu.VMEM((2, K), jnp.int32),
        pltpu.VMEM((2, K), jnp.int32),
        pltpu.VMEM((2, K, D), jnp.float32),
        pltpu.VMEM_SHARED((ACC_ROWS, D), jnp.float32),
        pltpu.SemaphoreType.DMA,
        pltpu.SemaphoreType.DMA,
    ],
    compiler_params=pltpu.CompilerParams(use_tc_tiling_on_sc=False),
)
def _sc_agg(y_hbm, pix_hbm, out_hbm,
            pix_v, gixb, dixb, rows_v, acc_sh, sem0, sem1):
    cid = lax.axis_index("c")
    sid = lax.axis_index("s")
    base = cid * HALF

    def zbody(j, c):
        rows_v[0, j // 16, pl.ds((j % 16) * 16, 16)] = jnp.zeros((16,),
                                                                 jnp.float32)
        return c

    lax.fori_loop(0, K * 16, zbody, 0)
    zb = sid * ZCHUNK
    for o in range(0, ZCHUNK - K + 1, K):
        pltpu.sync_copy(rows_v.at[0], acc_sh.at[pl.ds(zb + o, K)])
    rem = ZCHUNK % K
    if rem:
        pltpu.sync_copy(rows_v.at[0, pl.ds(0, rem)],
                        acc_sh.at[pl.ds(zb + ZCHUNK - rem, rem)])
    pltpu.sync_copy(pix_hbm.at[pl.ds(sid * BPT, BPT)], pix_v)
    plsc.subcore_barrier()

    sems = (sem0, sem1)

    def prep(b, slot):
        for w in range(K // 16):
            v = pix_v[b, pl.ds(w * 16, 16)]
            g = v & 0xFFFF
            d = lax.shift_right_logical(v, 16)
            inr = (d >= base) & (d < base + HALF)
            gixb[slot, pl.ds(w * 16, 16)] = g
            dixb[slot, pl.ds(w * 16, 16)] = jnp.where(inr, d - base, TRASH)
        pltpu.async_copy(y_hbm.at[gixb.at[slot]], rows_v.at[slot], sems[slot])

    def wait_scatter(slot):
        pltpu.make_async_copy(y_hbm.at[pl.ds(0, K)], rows_v.at[slot],
                              sems[slot]).wait()
        pltpu.sync_copy(rows_v.at[slot], acc_sh.at[dixb.at[slot]], add=True)

    npair = BPT // 2
    prep(0, 0)

    def body(g, c):
        b0 = 2 * g
        prep(b0 + 1, 1)
        wait_scatter(0)

        @pl.when(g + 1 < npair)
        def _():
            prep(b0 + 2, 0)

        wait_scatter(1)
        return c

    lax.fori_loop(0, npair, body, 0)
    plsc.subcore_barrier()
    start = jnp.minimum(sid * WCHUNK, HALF - WCHUNK)
    pltpu.sync_copy(acc_sh.at[pl.ds(start, WCHUNK)],
                    out_hbm.at[pl.ds(cid * HALF + start, WCHUNK)])


def _mlp_body(a_ref, w1_ref, b1_ref, w2_ref, b2_ref, h_ref, st_ref):
    i = pl.program_id(0)
    a = a_ref[...]
    h1 = lax.dot_general(a, w1_ref[...], (((1,), (1,)), ((), ())),
                         precision=lax.Precision.HIGHEST,
                         preferred_element_type=jnp.float32)
    h1 = jnp.maximum(h1 + b1_ref[...], 0.0)
    h = lax.dot_general(h1, w2_ref[...], (((1,), (1,)), ((), ())),
                        precision=lax.Precision.HIGHEST,
                        preferred_element_type=jnp.float32)
    h = h + b2_ref[...]
    h_ref[...] = h

    @pl.when(i == 0)
    def _():
        st_ref[...] = jnp.zeros_like(st_ref)

    st_ref[0:1, :] += jnp.sum(h, axis=0, keepdims=True)
    st_ref[1:2, :] += jnp.sum(h * h, axis=0, keepdims=True)


def _mlp(aggr, w1, b1, w2, b2):
    return pl.pallas_call(
        _mlp_body,
        grid=(N // MB,),
        in_specs=[
            pl.BlockSpec((MB, D), lambda i: (i, 0)),
            pl.BlockSpec((2 * D, D), lambda i: (0, 0)),
            pl.BlockSpec((1, 2 * D), lambda i: (0, 0)),
            pl.BlockSpec((D, 2 * D), lambda i: (0, 0)),
            pl.BlockSpec((1, D), lambda i: (0, 0)),
        ],
        out_specs=[
            pl.BlockSpec((MB, D), lambda i: (i, 0)),
            pl.BlockSpec((8, D), lambda i: (0, 0)),
        ],
        out_shape=[
            jax.ShapeDtypeStruct((N, D), jnp.float32),
            jax.ShapeDtypeStruct((8, D), jnp.float32),
        ],
    )(aggr, w1, b1, w2, b2)


def _norm_body(h_ref, st_ref, gam_ref, bet_ref, o_ref):
    mean = st_ref[0:1, :] * (1.0 / N)
    var = st_ref[1:2, :] * (1.0 / N) - mean * mean
    inv = lax.rsqrt(var + EPS)
    o_ref[...] = jnp.maximum(
        (h_ref[...] - mean) * inv * gam_ref[...] + bet_ref[...], 0.0)


def _norm(h, st, gamma, beta):
    return pl.pallas_call(
        _norm_body,
        grid=(N // MB,),
        in_specs=[
            pl.BlockSpec((MB, D), lambda i: (i, 0)),
            pl.BlockSpec((8, D), lambda i: (0, 0)),
            pl.BlockSpec((1, D), lambda i: (0, 0)),
            pl.BlockSpec((1, D), lambda i: (0, 0)),
        ],
        out_specs=pl.BlockSpec((MB, D), lambda i: (i, 0)),
        out_shape=jax.ShapeDtypeStruct((N, D), jnp.float32),
    )(h, st, gamma, beta)


def _process(x, edge_index, edge_attr, w_type, w1, b1, w2, b2, gamma, beta):
    ei = edge_index.astype(jnp.int32)
    t = edge_attr[:, 0].astype(jnp.int32)
    loops = jnp.arange(N, dtype=jnp.int32)
    npad = EPAD - E - N
    padz = jnp.zeros((npad,), jnp.int32)
    srcp = jnp.concatenate([ei[0], loops, padz]).reshape(EROWS, K)
    dstp = jnp.concatenate(
        [ei[1], loops, jnp.full((npad,), PADDST, jnp.int32)]).reshape(EROWS, K)
    typp = jnp.concatenate(
        [t, jnp.full((N,), 4, jnp.int32), padz]).reshape(EROWS, K)
    packed = _idx_prep(srcp, dstp, typp)
    y = _ybuild(x, w_type)
    aggr = _sc_agg(y, packed)
    h, st = _mlp(aggr, w1, b1.reshape(1, -1), w2, b2.reshape(1, -1))
    return _norm(h, st, gamma.reshape(1, -1), beta.reshape(1, -1))


def kernel(xA, edge_indexA, edge_attrA, xB, edge_indexB, edge_attrB,
           W_type, W1, b1, W2, b2, gamma, beta):
    outA = _process(xA, edge_indexA, edge_attrA, W_type, W1, b1, W2, b2,
                    gamma, beta)
    outB = _process(xB, edge_indexB, edge_attrB, W_type, W1, b1, W2, b2,
                    gamma, beta)
    return (outA, outB)
